# baseline (speedup 1.0000x reference)
import jax
import jax.numpy as jnp
from jax import lax
from jax.experimental import pallas as pl
from jax.experimental.pallas import tpu as pltpu
from jax.experimental.pallas import tpu_sc as plsc

_CUTS = (0, 20000, 100000, 1000000)
_D = 128
_SCALE = float(_D) ** 0.5

_NC, _NS = 2, 16
_NW = _NC * _NS
_CT = _CUTS[2] + 8

_B = 4096 * 200
_CB = 128
_BPW = _B // _NW
_NCH = _BPW // _CB


def _pt_kernel(emb_ref, proj_ref, e2row_ref, proj2_ref, out_ref):
    out = lax.dot_general(
        emb_ref[...], proj_ref[...] * _SCALE,
        (((1,), (1,)), ((), ())),
        preferred_element_type=jnp.float32)
    corr = lax.dot_general(
        e2row_ref[0:1, :], proj2_ref[...] * _SCALE,
        (((1,), (1,)), ((), ())),
        preferred_element_type=jnp.float32)
    out_ref[...] = out - corr


def _pt_kernel_aliased(tab_ref, emb_ref, proj_ref, e2row_ref, proj2_ref,
                       out_ref):
    del tab_ref
    _pt_kernel(emb_ref, proj_ref, e2row_ref, proj2_ref, out_ref)


def _zero_p2t_kernel(tab_ref, proj2_ref, zero_ref, p2t_ref):
    del tab_ref
    zero_ref[...] = jnp.zeros_like(zero_ref)
    eye8 = (lax.broadcasted_iota(jnp.int32, (8, 8), 0) ==
            lax.broadcasted_iota(jnp.int32, (8, 8), 1)).astype(jnp.float32)
    p2t_ref[...] = lax.dot_general(
        eye8, proj2_ref[...] * _SCALE,
        (((1,), (1,)), ((), ())),
        preferred_element_type=jnp.float32)


def _idx_kernel(inp_ref, g_ref, r_ref):
    x = inp_ref[...]
    g_ref[...] = jnp.minimum(x, _CUTS[2])
    r_ref[...] = jnp.maximum(x - _CUTS[2], 0)


def _build_tables(emb_0, emb_1, emb_2, proj_0, proj_1, proj_2):
    out_shape = jax.ShapeDtypeStruct((_CT, _D), jnp.float32)
    e2row_spec = pl.BlockSpec((8, 8), lambda i: (0, 0))
    br = 4000
    ctab = pl.pallas_call(
        _pt_kernel,
        grid=(_CUTS[1] // br,),
        in_specs=[pl.BlockSpec((br, 128), lambda i: (i, 0)),
                  pl.BlockSpec((128, 128), lambda i: (0, 0)),
                  e2row_spec,
                  pl.BlockSpec((128, 8), lambda i: (0, 0))],
        out_specs=pl.BlockSpec((br, _D), lambda i: (i, 0)),
        out_shape=out_shape,
    )(emb_0, proj_0, emb_2, proj_2)
    ctab = pl.pallas_call(
        _pt_kernel_aliased,
        grid=((_CUTS[2] - _CUTS[1]) // br,),
        in_specs=[pl.BlockSpec(memory_space=pl.ANY),
                  pl.BlockSpec((br, 32), lambda i: (i, 0)),
                  pl.BlockSpec((128, 32), lambda i: (0, 0)),
                  e2row_spec,
                  pl.BlockSpec((128, 8), lambda i: (0, 0))],
        out_specs=pl.BlockSpec((br, _D), lambda i: (i + _CUTS[1] // br, 0)),
        out_shape=out_shape,
        input_output_aliases={0: 0},
    )(ctab, emb_1, proj_1, emb_2, proj_2)
    ctab, p2t = pl.pallas_call(
        _zero_p2t_kernel,
        grid=(1,),
        in_specs=[pl.BlockSpec(memory_space=pl.ANY),
                  pl.BlockSpec((128, 8), lambda i: (0, 0))],
        out_specs=[pl.BlockSpec((8, _D), lambda i: (_CUTS[2] // 8, 0)),
                   pl.BlockSpec((8, _D), lambda i: (0, 0))],
        out_shape=[out_shape, jax.ShapeDtypeStruct((8, _D), jnp.float32)],
        input_output_aliases={0: 0},
    )(ctab, proj_2)
    return ctab, p2t


def _prep_idx(inp2d):
    shp = jax.ShapeDtypeStruct(inp2d.shape, jnp.int32)
    nb = 4
    brr = inp2d.shape[0] // nb
    return pl.pallas_call(
        _idx_kernel,
        grid=(nb,),
        in_specs=[pl.BlockSpec((brr, _CB), lambda i: (i, 0))],
        out_specs=[pl.BlockSpec((brr, _CB), lambda i: (i, 0)),
                   pl.BlockSpec((brr, _CB), lambda i: (i, 0))],
        out_shape=[shp, shp],
    )(inp2d)


def _splat(x):
    return jnp.full((16,), x, jnp.int32)


def _bcast_lane(v, k):
    dn = lax.GatherDimensionNumbers(offset_dims=(), collapsed_slice_dims=(0,),
                                    start_index_map=(0,))
    return lax.gather(v, _splat(k)[:, None], dn, (1,),
                      mode=lax.GatherScatterMode.PROMISE_IN_BOUNDS)


def _sc_body(ctab, e2p, p2t_hbm, idxg_hbm, idxr_hbm, out_hbm,
             idxg_v, idxr_v, rpbuf, gbuf, rbuf, p2t_v, rawT_v,
             semg0, semg1, semr0, semr1, semw0, semw1):
    wid = lax.axis_index("s") * _NC + lax.axis_index("c")
    base = wid * _BPW
    pltpu.sync_copy(idxg_hbm.at[pl.ds(wid * _NCH, _NCH)], idxg_v)
    pltpu.sync_copy(idxr_hbm.at[pl.ds(wid * _NCH, _NCH)], idxr_v)
    pltpu.sync_copy(p2t_hbm, p2t_v)

    def start(g, slot, semg, semr):
        pltpu.async_copy(ctab.at[idxg_v.at[g]], gbuf.at[slot], semg)
        for q in range(_CB // 16):
            rpbuf[slot, pl.ds(16 * q, 16)] = lax.shift_right_logical(
                idxr_v[g, pl.ds(16 * q, 16)], 1)
        pltpu.async_copy(e2p.at[rpbuf.at[slot]], rbuf.at[slot], semr)

    def wait_gr(slot, semg, semr):
        pltpu.make_async_copy(ctab.at[idxg_v.at[0]], gbuf.at[slot],
                              semg).wait()
        pltpu.make_async_copy(e2p.at[rpbuf.at[slot]], rbuf.at[slot],
                              semr).wait()

    def putw(g, slot, semw):
        pltpu.async_copy(gbuf.at[slot],
                         out_hbm.at[pl.ds(base + g * _CB, _CB)], semw)

    def wait_w(slot, semw):
        pltpu.make_async_copy(gbuf.at[slot], out_hbm.at[pl.ds(base, _CB)],
                              semw).wait()

    lanes = lax.broadcasted_iota(jnp.int32, (16,), 0)

    def compute(g, slot):
        for grp in range(_CB // 16):
            tok = lanes + 16 * grp
            par = jnp.bitwise_and(idxr_v[g, pl.ds(16 * grp, 16)], 1) * 8
            for k in range(8):
                vals = plsc.load_gather(
                    rbuf, [_splat(slot), tok, par + k])
                rawT_v[k, pl.ds(16 * grp, 16)] = vals
        for half in range(2):
            rT = [[rawT_v[k, pl.ds(64 * half + 16 * grp, 16)]
                   for grp in range(4)] for k in range(8)]

            def dbody(d, carry):
                dv = _splat(0) + d
                bv = [plsc.load_gather(p2t_v, [_splat(k), dv])
                      for k in range(8)]
                for grp in range(4):
                    tok = lanes + 64 * half + 16 * grp
                    acc = plsc.load_gather(gbuf, [_splat(slot), tok, dv])
                    for k in range(8):
                        acc = acc + rT[k][grp] * bv[k]
                    plsc.store_scatter(gbuf, [_splat(slot), tok, dv], acc)
                return carry

            lax.fori_loop(0, _D, dbody, 0)

    start(0, 0, semg0, semr0)

    def body(p, carry):
        g0 = 2 * p
        start(g0 + 1, 1, semg1, semr1)
        wait_gr(0, semg0, semr0)
        compute(g0, 0)
        putw(g0, 0, semw0)
        wait_gr(1, semg1, semr1)
        compute(g0 + 1, 1)

        @pl.when(g0 + 2 < _NCH)
        def _():
            wait_w(0, semw0)
            start(g0 + 2, 0, semg0, semr0)

        putw(g0 + 1, 1, semw1)
        wait_w(1, semw1)
        return carry

    lax.fori_loop(0, _NCH // 2, body, 0)
    wait_w(0, semw0)


_sc_lookup = pl.kernel(
    _sc_body,
    out_type=jax.ShapeDtypeStruct((_B, _D), jnp.float32),
    mesh=plsc.VectorSubcoreMesh(core_axis_name="c", subcore_axis_name="s"),
    compiler_params=pltpu.CompilerParams(use_tc_tiling_on_sc=False,
                                         needs_layout_passes=False),
    scratch_types=[
        pltpu.VMEM((_NCH, _CB), jnp.int32),
        pltpu.VMEM((_NCH, _CB), jnp.int32),
        pltpu.VMEM((2, _CB), jnp.int32),
        pltpu.VMEM((2, _CB, _D), jnp.float32),
        pltpu.VMEM((2, _CB, 16), jnp.float32),
        pltpu.VMEM((8, _D), jnp.float32),
        pltpu.VMEM((8, _CB), jnp.float32),
        pltpu.SemaphoreType.DMA,
        pltpu.SemaphoreType.DMA,
        pltpu.SemaphoreType.DMA,
        pltpu.SemaphoreType.DMA,
        pltpu.SemaphoreType.DMA,
        pltpu.SemaphoreType.DMA,
    ],
)


def kernel(inp, emb_0, emb_1, emb_2, proj_0, proj_1, proj_2):
    ctab, p2t = _build_tables(emb_0, emb_1, emb_2, proj_0, proj_1, proj_2)
    idxg, idxr = _prep_idx(inp.reshape(_B // _CB, _CB))
    e2p = emb_2.reshape(_CUTS[3] // 2 - _CUTS[2] // 2, 16)
    out = _sc_lookup(ctab, e2p, p2t, idxg, idxr)
    return out.reshape(inp.shape[0], inp.shape[1], _D)


# speedup vs baseline: 4.3489x; 4.3489x over previous
import jax
import jax.numpy as jnp
from jax import lax
from jax.experimental import pallas as pl
from jax.experimental.pallas import tpu as pltpu
from jax.experimental.pallas import tpu_sc as plsc

_CUTS = (0, 20000, 100000, 1000000)
_D = 128
_SCALE = float(_D) ** 0.5

_NC, _NS = 2, 16
_NW = _NC * _NS
_CT = _CUTS[2] + 8

_B = 4096 * 200
_CB = 128
_BPW = _B // _NW
_NCH = _BPW // _CB


def _pt_kernel(emb_ref, proj_ref, e2row_ref, proj2_ref, out_ref):
    out = lax.dot_general(
        emb_ref[...], proj_ref[...] * _SCALE,
        (((1,), (1,)), ((), ())),
        preferred_element_type=jnp.float32)
    corr = lax.dot_general(
        e2row_ref[0:1, :], proj2_ref[...] * _SCALE,
        (((1,), (1,)), ((), ())),
        preferred_element_type=jnp.float32)
    out_ref[...] = out - corr


def _pt_kernel_aliased(tab_ref, emb_ref, proj_ref, e2row_ref, proj2_ref,
                       out_ref):
    del tab_ref
    _pt_kernel(emb_ref, proj_ref, e2row_ref, proj2_ref, out_ref)


def _zero_p2t_kernel(tab_ref, proj2_ref, zero_ref, p2t_ref):
    del tab_ref
    zero_ref[...] = jnp.zeros_like(zero_ref)
    eye8 = (lax.broadcasted_iota(jnp.int32, (8, 8), 0) ==
            lax.broadcasted_iota(jnp.int32, (8, 8), 1)).astype(jnp.float32)
    p2t_ref[...] = lax.dot_general(
        eye8, proj2_ref[...] * _SCALE,
        (((1,), (1,)), ((), ())),
        preferred_element_type=jnp.float32)


def _idx_kernel(inp_ref, g_ref, r_ref, m_ref):
    x = inp_ref[...]
    is01 = x < _CUTS[2]
    g_ref[...] = jnp.where(is01, x, jnp.bitwise_and(x, 65535))
    r_ref[...] = jnp.maximum(x - _CUTS[2], 0)
    m_ref[...] = is01.astype(jnp.float32)


def _build_tables(emb_0, emb_1, emb_2, proj_0, proj_1, proj_2):
    out_shape = jax.ShapeDtypeStruct((_CT, _D), jnp.float32)
    e2row_spec = pl.BlockSpec((8, 8), lambda i: (0, 0))
    br = 4000
    ctab = pl.pallas_call(
        _pt_kernel,
        grid=(_CUTS[1] // br,),
        in_specs=[pl.BlockSpec((br, 128), lambda i: (i, 0)),
                  pl.BlockSpec((128, 128), lambda i: (0, 0)),
                  e2row_spec,
                  pl.BlockSpec((128, 8), lambda i: (0, 0))],
        out_specs=pl.BlockSpec((br, _D), lambda i: (i, 0)),
        out_shape=out_shape,
    )(emb_0, proj_0, emb_2, proj_2)
    ctab = pl.pallas_call(
        _pt_kernel_aliased,
        grid=((_CUTS[2] - _CUTS[1]) // br,),
        in_specs=[pl.BlockSpec(memory_space=pl.ANY),
                  pl.BlockSpec((br, 32), lambda i: (i, 0)),
                  pl.BlockSpec((128, 32), lambda i: (0, 0)),
                  e2row_spec,
                  pl.BlockSpec((128, 8), lambda i: (0, 0))],
        out_specs=pl.BlockSpec((br, _D), lambda i: (i + _CUTS[1] // br, 0)),
        out_shape=out_shape,
        input_output_aliases={0: 0},
    )(ctab, emb_1, proj_1, emb_2, proj_2)
    ctab, p2t = pl.pallas_call(
        _zero_p2t_kernel,
        grid=(1,),
        in_specs=[pl.BlockSpec(memory_space=pl.ANY),
                  pl.BlockSpec((128, 8), lambda i: (0, 0))],
        out_specs=[pl.BlockSpec((8, _D), lambda i: (_CUTS[2] // 8, 0)),
                   pl.BlockSpec((8, _D), lambda i: (0, 0))],
        out_shape=[out_shape, jax.ShapeDtypeStruct((8, _D), jnp.float32)],
        input_output_aliases={0: 0},
    )(ctab, proj_2)
    return ctab, p2t


def _prep_idx(inp2d):
    shp = jax.ShapeDtypeStruct(inp2d.shape, jnp.int32)
    shpf = jax.ShapeDtypeStruct(inp2d.shape, jnp.float32)
    nb = 4
    brr = inp2d.shape[0] // nb
    return pl.pallas_call(
        _idx_kernel,
        grid=(nb,),
        in_specs=[pl.BlockSpec((brr, _CB), lambda i: (i, 0))],
        out_specs=[pl.BlockSpec((brr, _CB), lambda i: (i, 0)),
                   pl.BlockSpec((brr, _CB), lambda i: (i, 0)),
                   pl.BlockSpec((brr, _CB), lambda i: (i, 0))],
        out_shape=[shp, shp, shpf],
    )(inp2d)


def _splat(x):
    return jnp.full((16,), x, jnp.int32)


def _bcast_lane(v, k):
    dn = lax.GatherDimensionNumbers(offset_dims=(), collapsed_slice_dims=(0,),
                                    start_index_map=(0,))
    return lax.gather(v, _splat(k)[:, None], dn, (1,),
                      mode=lax.GatherScatterMode.PROMISE_IN_BOUNDS)


def _sc_body(ctab, e2p, p2t_hbm, idxg_hbm, idxr_hbm, idxm_hbm, out_hbm,
             idxg_v, idxr_v, idxm_v, rpbuf, gbuf, rbuf, p2t_v, rawT_v,
             semg0, semg1, semr0, semr1, semw0, semw1):
    wid = lax.axis_index("s") * _NC + lax.axis_index("c")
    base = wid * _BPW
    pltpu.sync_copy(idxg_hbm.at[pl.ds(wid * _NCH, _NCH)], idxg_v)
    pltpu.sync_copy(idxr_hbm.at[pl.ds(wid * _NCH, _NCH)], idxr_v)
    pltpu.sync_copy(idxm_hbm.at[pl.ds(wid * _NCH, _NCH)], idxm_v)
    pltpu.sync_copy(p2t_hbm, p2t_v)

    def start(g, slot, semg, semr):
        pltpu.async_copy(ctab.at[idxg_v.at[g]], gbuf.at[slot], semg)
        for q in range(_CB // 16):
            rpbuf[slot, pl.ds(16 * q, 16)] = lax.shift_right_logical(
                idxr_v[g, pl.ds(16 * q, 16)], 1)
        pltpu.async_copy(e2p.at[rpbuf.at[slot]], rbuf.at[slot], semr)

    def wait_gr(slot, semg, semr):
        pltpu.make_async_copy(ctab.at[idxg_v.at[0]], gbuf.at[slot],
                              semg).wait()
        pltpu.make_async_copy(e2p.at[rpbuf.at[slot]], rbuf.at[slot],
                              semr).wait()

    def putw(g, slot, semw):
        pltpu.async_copy(gbuf.at[slot],
                         out_hbm.at[pl.ds(base + g * _CB, _CB)], semw)

    def wait_w(slot, semw):
        pltpu.make_async_copy(gbuf.at[slot], out_hbm.at[pl.ds(base, _CB)],
                              semw).wait()

    lanes = lax.broadcasted_iota(jnp.int32, (16,), 0)

    def compute(g, slot):
        for grp in range(_CB // 16):
            tok = lanes + 16 * grp
            par = jnp.bitwise_and(idxr_v[g, pl.ds(16 * grp, 16)], 1) * 8
            for k in range(8):
                vals = plsc.load_gather(
                    rbuf, [_splat(slot), tok, par + k])
                rawT_v[k, pl.ds(16 * grp, 16)] = vals
        for half in range(2):
            rT = [[rawT_v[k, pl.ds(64 * half + 16 * grp, 16)]
                   for grp in range(4)] for k in range(8)]
            mv = [idxm_v[g, pl.ds(64 * half + 16 * grp, 16)]
                  for grp in range(4)]

            def dbody(d, carry):
                dv = _splat(0) + d
                bv = [plsc.load_gather(p2t_v, [_splat(k), dv])
                      for k in range(8)]
                for grp in range(4):
                    tok = lanes + 64 * half + 16 * grp
                    acc = plsc.load_gather(gbuf, [_splat(slot), tok, dv])
                    acc = acc * mv[grp]
                    for k in range(8):
                        acc = acc + rT[k][grp] * bv[k]
                    plsc.store_scatter(gbuf, [_splat(slot), tok, dv], acc)
                return carry

            lax.fori_loop(0, _D, dbody, 0)

    start(0, 0, semg0, semr0)

    def body(p, carry):
        g0 = 2 * p
        start(g0 + 1, 1, semg1, semr1)
        wait_gr(0, semg0, semr0)
        compute(g0, 0)
        putw(g0, 0, semw0)
        wait_gr(1, semg1, semr1)
        compute(g0 + 1, 1)

        @pl.when(g0 + 2 < _NCH)
        def _():
            wait_w(0, semw0)
            start(g0 + 2, 0, semg0, semr0)

        putw(g0 + 1, 1, semw1)
        wait_w(1, semw1)
        return carry

    lax.fori_loop(0, _NCH // 2, body, 0)
    wait_w(0, semw0)


_sc_lookup = pl.kernel(
    _sc_body,
    out_type=jax.ShapeDtypeStruct((_B, _D), jnp.float32),
    mesh=plsc.VectorSubcoreMesh(core_axis_name="c", subcore_axis_name="s"),
    compiler_params=pltpu.CompilerParams(use_tc_tiling_on_sc=False,
                                         needs_layout_passes=False),
    scratch_types=[
        pltpu.VMEM((_NCH, _CB), jnp.int32),
        pltpu.VMEM((_NCH, _CB), jnp.int32),
        pltpu.VMEM((_NCH, _CB), jnp.float32),
        pltpu.VMEM((2, _CB), jnp.int32),
        pltpu.VMEM((2, _CB, _D), jnp.float32),
        pltpu.VMEM((2, _CB, 16), jnp.float32),
        pltpu.VMEM((8, _D), jnp.float32),
        pltpu.VMEM((8, _CB), jnp.float32),
        pltpu.SemaphoreType.DMA,
        pltpu.SemaphoreType.DMA,
        pltpu.SemaphoreType.DMA,
        pltpu.SemaphoreType.DMA,
        pltpu.SemaphoreType.DMA,
        pltpu.SemaphoreType.DMA,
    ],
)


def kernel(inp, emb_0, emb_1, emb_2, proj_0, proj_1, proj_2):
    ctab, p2t = _build_tables(emb_0, emb_1, emb_2, proj_0, proj_1, proj_2)
    idxg, idxr, idxm = _prep_idx(inp.reshape(_B // _CB, _CB))
    e2p = emb_2.reshape(_CUTS[3] // 2 - _CUTS[2] // 2, 16)
    out = _sc_lookup(ctab, e2p, p2t, idxg, idxr, idxm)
    return out.reshape(inp.shape[0], inp.shape[1], _D)


# speedup vs baseline: 33.5184x; 7.7073x over previous
import functools

import jax
import jax.numpy as jnp
from jax import lax
from jax.experimental import pallas as pl
from jax.experimental.pallas import tpu as pltpu
from jax.experimental.pallas import tpu_sc as plsc

_CUTS = (0, 20000, 100000, 1000000)
_D = 128
_SCALE = float(_D) ** 0.5

_NC, _NS = 2, 16
_NW = _NC * _NS
_V = _CUTS[3]
_BR = 2000

_B = 4096 * 200
_CB = 128
_BPW = _B // _NW
_NCH = _BPW // _CB


def _proj_kernel(emb_ref, proj_ref, out_ref):
    out_ref[...] = lax.dot_general(
        emb_ref[...], proj_ref[...] * _SCALE,
        (((1,), (1,)), ((), ())),
        preferred_element_type=jnp.float32)


def _proj_kernel_aliased(tab_ref, emb_ref, proj_ref, out_ref):
    del tab_ref
    _proj_kernel(emb_ref, proj_ref, out_ref)


def _build_table(emb_0, emb_1, emb_2, proj_0, proj_1, proj_2):
    out_shape = jax.ShapeDtypeStruct((_V, _D), jnp.float32)
    br0 = 10000
    table = pl.pallas_call(
        _proj_kernel,
        grid=(_CUTS[1] // br0,),
        in_specs=[pl.BlockSpec((br0, 128), lambda i: (i, 0)),
                  pl.BlockSpec((128, 128), lambda i: (0, 0))],
        out_specs=pl.BlockSpec((br0, _D), lambda i: (i, 0)),
        out_shape=out_shape,
    )(emb_0, proj_0)
    for emb, proj, lo, hi, br in (
            (emb_1, proj_1, _CUTS[1], _CUTS[2], 10000),
            (emb_2, proj_2, _CUTS[2], _CUTS[3], 25000)):
        off = lo // br
        d_in = emb.shape[1]
        table = pl.pallas_call(
            _proj_kernel_aliased,
            grid=((hi - lo) // br,),
            in_specs=[pl.BlockSpec(memory_space=pl.ANY),
                      pl.BlockSpec((br, d_in), lambda i: (i, 0)),
                      pl.BlockSpec((128, d_in), lambda i: (0, 0))],
            out_specs=pl.BlockSpec((br, _D),
                                   lambda i, _off=off: (i + _off, 0)),
            out_shape=out_shape,
            input_output_aliases={0: 0},
        )(table, emb, proj)
    return table


def _sc_lookup_body(table_hbm, idx_hbm, out_hbm, idx_v, rows_v, sem0, sem1):
    wid = lax.axis_index("s") * _NC + lax.axis_index("c")
    base = wid * _BPW
    pltpu.sync_copy(idx_hbm.at[pl.ds(wid * _NCH, _NCH)], idx_v)

    def start(g, slot, sem):
        pltpu.async_copy(table_hbm.at[idx_v.at[g]], rows_v.at[slot], sem)

    def drain(slot, sem):
        pltpu.make_async_copy(table_hbm.at[idx_v.at[0]], rows_v.at[slot],
                              sem).wait()

    def put(g, slot):
        pltpu.sync_copy(rows_v.at[slot], out_hbm.at[pl.ds(base + g * _CB, _CB)])

    start(0, 0, sem0)

    def body(p, carry):
        g0 = 2 * p
        start(g0 + 1, 1, sem1)
        drain(0, sem0)
        put(g0, 0)

        @pl.when(g0 + 2 < _NCH)
        def _():
            start(g0 + 2, 0, sem0)

        drain(1, sem1)
        put(g0 + 1, 1)
        return carry

    lax.fori_loop(0, _NCH // 2, body, 0)


_sc_lookup = pl.kernel(
    _sc_lookup_body,
    out_type=jax.ShapeDtypeStruct((_B, _D), jnp.float32),
    mesh=plsc.VectorSubcoreMesh(core_axis_name="c", subcore_axis_name="s"),
    scratch_types=[
        pltpu.VMEM((_NCH, _CB), jnp.int32),
        pltpu.VMEM((2, _CB, _D), jnp.float32),
        pltpu.SemaphoreType.DMA,
        pltpu.SemaphoreType.DMA,
    ],
)


def kernel(inp, emb_0, emb_1, emb_2, proj_0, proj_1, proj_2):
    table = _build_table(emb_0, emb_1, emb_2, proj_0, proj_1, proj_2)
    idx2d = inp.reshape(_B // _CB, _CB)
    out = _sc_lookup(table, idx2d)
    return out.reshape(inp.shape[0], inp.shape[1], _D)
